# Initial kernel scaffold; baseline (speedup 1.0000x reference)
import jax
import jax.numpy as jnp
from jax.experimental import pallas as pl


def kernel(x, edge_index, W1, b1, W2, b2, W3, b3, Wp, bp):
    raise NotImplementedError("write your pallas kernel here")


# speedup vs baseline: 13.1104x; 13.1104x over previous
import functools

import jax
import jax.numpy as jnp
from jax import lax
from jax.experimental import pallas as pl
from jax.experimental.pallas import tpu as pltpu
from jax.experimental.pallas import tpu_sc as plsc

_N = 10000
_E = 320000
_D = 128
_H = 20
_C = 10

_NTILES = 32
_NSUB = 16
_NP = 10112
_RPT = _NP // _NSUB
_CH = 128
_EP = 327680
_CPT = _EP // (_NTILES * _CH)


def _make_edge_pass(P):
  mesh = plsc.VectorSubcoreMesh(core_axis_name="c", subcore_axis_name="s")

  @functools.partial(
      pl.kernel,
      out_type=jax.ShapeDtypeStruct((2, _NP, P), jnp.float32),
      mesh=mesh,
      scratch_types=[
          pltpu.VMEM((_CPT, _CH), jnp.int32),
          pltpu.VMEM((_CPT, _CH), jnp.int32),
          pltpu.VMEM((_CH, P), jnp.float32),
          pltpu.VMEM_SHARED((_NP, P), jnp.float32),
          pltpu.SemaphoreType.DMA,
      ],
      compiler_params=pltpu.CompilerParams(use_tc_tiling_on_sc=False),
  )
  def edge_pass(table_hbm, sidx_hbm, didx_hbm, zeros_hbm, out_hbm,
                sidx_v, didx_v, rows_v, acc, sem):
    c = lax.axis_index("c")
    s = lax.axis_index("s")
    wid = c * _NSUB + s
    pltpu.sync_copy(zeros_hbm.at[pl.ds(s * _RPT, _RPT)],
                    acc.at[pl.ds(s * _RPT, _RPT)])
    pltpu.sync_copy(sidx_hbm.at[pl.ds(wid * _CPT, _CPT)], sidx_v)
    pltpu.sync_copy(didx_hbm.at[pl.ds(wid * _CPT, _CPT)], didx_v)
    plsc.subcore_barrier()

    def body(j, carry):
      pltpu.async_copy(table_hbm.at[sidx_v.at[j]], rows_v, sem).wait()
      pltpu.sync_copy(rows_v, acc.at[didx_v.at[j]], add=True)
      return carry

    lax.fori_loop(0, _CPT, body, 0)
    plsc.subcore_barrier()
    pltpu.sync_copy(acc.at[pl.ds(s * _RPT, _RPT)],
                    out_hbm.at[c].at[pl.ds(s * _RPT, _RPT)])

  return edge_pass


_edge_pass16 = _make_edge_pass(16)
_edge_pass32 = _make_edge_pass(32)


def _dinv_col(deg2_ref):
  d = deg2_ref[0] + deg2_ref[1]
  deg = d[:, 0:1]
  return jnp.where(deg > 0, lax.rsqrt(jnp.maximum(deg, 1e-12)), 0.0)


def _mm_body(a_ref, w_ref, o_ref):
  o_ref[...] = jnp.dot(a_ref[...], w_ref[...],
                       preferred_element_type=jnp.float32)


def _mm(a, w):
  return pl.pallas_call(
      _mm_body,
      out_shape=jax.ShapeDtypeStruct((a.shape[0], w.shape[1]), jnp.float32),
  )(a, w)


def _scale_body(deg2_ref, h_ref, o_ref):
  o_ref[...] = _dinv_col(deg2_ref) * h_ref[...]


def _scale(deg2, h):
  return pl.pallas_call(
      _scale_body,
      out_shape=jax.ShapeDtypeStruct(h.shape, jnp.float32),
  )(deg2, h)


def _layer_body(deg2_ref, acc_ref, b_ref, w_ref, h_ref, g_ref):
  dinv = _dinv_col(deg2_ref)
  su = acc_ref[0] + acc_ref[1]
  h = jnp.maximum(dinv * su + b_ref[...], 0.0)
  h_ref[...] = h
  g_ref[...] = dinv * jnp.dot(h, w_ref[...],
                              preferred_element_type=jnp.float32)


def _layer(deg2, acc, b, w):
  return pl.pallas_call(
      _layer_body,
      out_shape=(jax.ShapeDtypeStruct((_NP, 32), jnp.float32),
                 jax.ShapeDtypeStruct((_NP, 32), jnp.float32)),
  )(deg2, acc, b, w)


def _final_body(deg2_ref, acc_ref, b_ref, h1_ref, h2_ref, wp_ref, bp_ref,
                o_ref):
  dinv = _dinv_col(deg2_ref)
  su = acc_ref[0] + acc_ref[1]
  h3 = jnp.maximum(dinv * su + b_ref[...], 0.0)
  hcat = jnp.concatenate([h1_ref[...], h2_ref[...], h3], axis=1)
  logits = jnp.dot(hcat, wp_ref[...],
                   preferred_element_type=jnp.float32) + bp_ref[...]
  m = jnp.max(logits, axis=1, keepdims=True)
  lse = jnp.log(jnp.sum(jnp.exp(logits - m), axis=1, keepdims=True))
  o_ref[...] = logits - m - lse


def _final(deg2, acc, b, h1, h2, wp, bp):
  return pl.pallas_call(
      _final_body,
      out_shape=jax.ShapeDtypeStruct((_NP, 16), jnp.float32),
  )(deg2, acc, b, h1, h2, wp, bp)


def kernel(x, edge_index, W1, b1, W2, b2, W3, b3, Wp, bp):
  f32 = jnp.float32
  src = edge_index[0]
  dst = edge_index[1]
  pad = jnp.full((_EP - _E,), _N, jnp.int32)
  srcp = jnp.concatenate([src, pad]).reshape(_EP // _CH, _CH)
  dstp = jnp.concatenate([dst, pad]).reshape(_EP // _CH, _CH)

  xp = jnp.pad(x, ((0, _NP - _N), (0, 0)))
  W1p = jnp.pad(W1, ((0, 0), (0, 32 - _H)))
  W2p = jnp.pad(W2, ((0, 32 - _H), (0, 32 - _H)))
  W3p = jnp.pad(W3, ((0, 32 - _H), (0, 32 - _H)))
  b1p = jnp.pad(b1, (0, 32 - _H)).reshape(1, 32)
  b2p = jnp.pad(b2, (0, 32 - _H)).reshape(1, 32)
  b3p = jnp.pad(b3, (0, 32 - _H)).reshape(1, 32)
  wpp = (jnp.zeros((96, 16), f32)
         .at[0:_H, 0:_C].set(Wp[0:_H])
         .at[32:32 + _H, 0:_C].set(Wp[_H:2 * _H])
         .at[64:64 + _H, 0:_C].set(Wp[2 * _H:3 * _H]))
  bpp = jnp.full((1, 16), -1e30, f32).at[0, 0:_C].set(bp)

  zeros16 = jnp.zeros((_NP, 16), f32)
  zeros32 = jnp.zeros((_NP, 32), f32)
  ones16 = jnp.ones((_NP, 16), f32)

  deg2 = _edge_pass16(ones16, srcp, dstp, zeros16)
  h1p = _mm(xp, W1p)
  g1 = _scale(deg2, h1p)
  a1 = _edge_pass32(g1, srcp, dstp, zeros32)
  h1, g2 = _layer(deg2, a1, b1p, W2p)
  a2 = _edge_pass32(g2, srcp, dstp, zeros32)
  h2, g3 = _layer(deg2, a2, b2p, W3p)
  a3 = _edge_pass32(g3, srcp, dstp, zeros32)
  out = _final(deg2, a3, b3p, h1, h2, wpp, bpp)
  return out[:_N, :_C]


# speedup vs baseline: 16.9823x; 1.2953x over previous
import functools

import jax
import jax.numpy as jnp
from jax import lax
from jax.experimental import pallas as pl
from jax.experimental.pallas import tpu as pltpu
from jax.experimental.pallas import tpu_sc as plsc

_N = 10000
_E = 320000
_D = 128
_H = 20
_C = 10

_NTILES = 32
_NSUB = 16
_NP = 10112
_RPT = _NP // _NSUB
_CH = 128
_EP = 327680
_CPT = _EP // (_NTILES * _CH)


def _make_edge_pass(P):
  mesh = plsc.VectorSubcoreMesh(core_axis_name="c", subcore_axis_name="s")

  @functools.partial(
      pl.kernel,
      out_type=jax.ShapeDtypeStruct((2, _NP, P), jnp.float32),
      mesh=mesh,
      scratch_types=[
          pltpu.VMEM((_CPT, _CH), jnp.int32),
          pltpu.VMEM((_CPT, _CH), jnp.int32),
          pltpu.VMEM((_CH, P), jnp.float32),
          pltpu.VMEM((_CH, P), jnp.float32),
          pltpu.VMEM_SHARED((_NP, P), jnp.float32),
          pltpu.SemaphoreType.DMA,
          pltpu.SemaphoreType.DMA,
      ],
      compiler_params=pltpu.CompilerParams(use_tc_tiling_on_sc=False),
  )
  def edge_pass(table_hbm, sidx_hbm, didx_hbm, zeros_hbm, out_hbm,
                sidx_v, didx_v, rows_a, rows_b, acc, sem_a, sem_b):
    c = lax.axis_index("c")
    s = lax.axis_index("s")
    wid = c * _NSUB + s
    pltpu.sync_copy(zeros_hbm.at[pl.ds(s * _RPT, _RPT)],
                    acc.at[pl.ds(s * _RPT, _RPT)])
    pltpu.sync_copy(sidx_hbm.at[pl.ds(wid * _CPT, _CPT)], sidx_v)
    pltpu.sync_copy(didx_hbm.at[pl.ds(wid * _CPT, _CPT)], didx_v)
    plsc.subcore_barrier()

    nhalf = _CPT // 2
    pltpu.async_copy(table_hbm.at[sidx_v.at[0]], rows_a, sem_a)

    def body(i, carry):
      j0 = 2 * i
      pltpu.async_copy(table_hbm.at[sidx_v.at[j0 + 1]], rows_b, sem_b)
      pltpu.make_async_copy(table_hbm.at[sidx_v.at[0]], rows_a, sem_a).wait()
      pltpu.sync_copy(rows_a, acc.at[didx_v.at[j0]], add=True)

      @pl.when(i < nhalf - 1)
      def _():
        pltpu.async_copy(table_hbm.at[sidx_v.at[j0 + 2]], rows_a, sem_a)

      pltpu.make_async_copy(table_hbm.at[sidx_v.at[0]], rows_b, sem_b).wait()
      pltpu.sync_copy(rows_b, acc.at[didx_v.at[j0 + 1]], add=True)
      return carry

    lax.fori_loop(0, nhalf, body, 0)
    plsc.subcore_barrier()
    pltpu.sync_copy(acc.at[pl.ds(s * _RPT, _RPT)],
                    out_hbm.at[c].at[pl.ds(s * _RPT, _RPT)])

  return edge_pass


_edge_pass16 = _make_edge_pass(16)
_edge_pass32 = _make_edge_pass(32)


def _dinv_col(deg2_ref):
  d = deg2_ref[0] + deg2_ref[1]
  deg = d[:, 0:1]
  return jnp.where(deg > 0, lax.rsqrt(jnp.maximum(deg, 1e-12)), 0.0)


def _mm_body(a_ref, w_ref, o_ref):
  o_ref[...] = jnp.dot(a_ref[...], w_ref[...],
                       preferred_element_type=jnp.float32)


def _mm(a, w):
  return pl.pallas_call(
      _mm_body,
      out_shape=jax.ShapeDtypeStruct((a.shape[0], w.shape[1]), jnp.float32),
  )(a, w)


def _scale_body(deg2_ref, h_ref, o_ref):
  o_ref[...] = _dinv_col(deg2_ref) * h_ref[...]


def _scale(deg2, h):
  return pl.pallas_call(
      _scale_body,
      out_shape=jax.ShapeDtypeStruct(h.shape, jnp.float32),
  )(deg2, h)


def _layer_body(deg2_ref, acc_ref, b_ref, w_ref, h_ref, g_ref):
  dinv = _dinv_col(deg2_ref)
  su = acc_ref[0] + acc_ref[1]
  h = jnp.maximum(dinv * su + b_ref[...], 0.0)
  h_ref[...] = h
  g_ref[...] = dinv * jnp.dot(h, w_ref[...],
                              preferred_element_type=jnp.float32)


def _layer(deg2, acc, b, w):
  return pl.pallas_call(
      _layer_body,
      out_shape=(jax.ShapeDtypeStruct((_NP, 32), jnp.float32),
                 jax.ShapeDtypeStruct((_NP, 32), jnp.float32)),
  )(deg2, acc, b, w)


def _final_body(deg2_ref, acc_ref, b_ref, h1_ref, h2_ref, wp_ref, bp_ref,
                o_ref):
  dinv = _dinv_col(deg2_ref)
  su = acc_ref[0] + acc_ref[1]
  h3 = jnp.maximum(dinv * su + b_ref[...], 0.0)
  hcat = jnp.concatenate([h1_ref[...], h2_ref[...], h3], axis=1)
  logits = jnp.dot(hcat, wp_ref[...],
                   preferred_element_type=jnp.float32) + bp_ref[...]
  m = jnp.max(logits, axis=1, keepdims=True)
  lse = jnp.log(jnp.sum(jnp.exp(logits - m), axis=1, keepdims=True))
  o_ref[...] = logits - m - lse


def _final(deg2, acc, b, h1, h2, wp, bp):
  return pl.pallas_call(
      _final_body,
      out_shape=jax.ShapeDtypeStruct((_NP, 16), jnp.float32),
  )(deg2, acc, b, h1, h2, wp, bp)


def kernel(x, edge_index, W1, b1, W2, b2, W3, b3, Wp, bp):
  f32 = jnp.float32
  src = edge_index[0]
  dst = edge_index[1]
  pad = jnp.full((_EP - _E,), _N, jnp.int32)
  srcp = jnp.concatenate([src, pad]).reshape(_EP // _CH, _CH)
  dstp = jnp.concatenate([dst, pad]).reshape(_EP // _CH, _CH)

  xp = jnp.pad(x, ((0, _NP - _N), (0, 0)))
  W1p = jnp.pad(W1, ((0, 0), (0, 32 - _H)))
  W2p = jnp.pad(W2, ((0, 32 - _H), (0, 32 - _H)))
  W3p = jnp.pad(W3, ((0, 32 - _H), (0, 32 - _H)))
  b1p = jnp.pad(b1, (0, 32 - _H)).reshape(1, 32)
  b2p = jnp.pad(b2, (0, 32 - _H)).reshape(1, 32)
  b3p = jnp.pad(b3, (0, 32 - _H)).reshape(1, 32)
  wpp = (jnp.zeros((96, 16), f32)
         .at[0:_H, 0:_C].set(Wp[0:_H])
         .at[32:32 + _H, 0:_C].set(Wp[_H:2 * _H])
         .at[64:64 + _H, 0:_C].set(Wp[2 * _H:3 * _H]))
  bpp = jnp.full((1, 16), -1e30, f32).at[0, 0:_C].set(bp)

  zeros16 = jnp.zeros((_NP, 16), f32)
  zeros32 = jnp.zeros((_NP, 32), f32)
  ones16 = jnp.ones((_NP, 16), f32)

  deg2 = _edge_pass16(ones16, srcp, dstp, zeros16)
  h1p = _mm(xp, W1p)
  g1 = _scale(deg2, h1p)
  a1 = _edge_pass32(g1, srcp, dstp, zeros32)
  h1, g2 = _layer(deg2, a1, b1p, W2p)
  a2 = _edge_pass32(g2, srcp, dstp, zeros32)
  h2, g3 = _layer(deg2, a2, b2p, W3p)
  a3 = _edge_pass32(g3, srcp, dstp, zeros32)
  out = _final(deg2, a3, b3p, h1, h2, wpp, bpp)
  return out[:_N, :_C]
